# baseline (speedup 1.0000x reference)
import functools

import jax
import jax.numpy as jnp
from jax import lax
from jax.experimental import pallas as pl
from jax.experimental.pallas import tpu as pltpu
from jax.experimental.pallas import tpu_sc as plsc

S, W, B, V, D = 128, 16, 256, 1000, 64
L = 16
BC = B // L
SEQ_PER_TILE = 4
NC = D // L

_mesh = plsc.VectorSubcoreMesh(core_axis_name="c", subcore_axis_name="s")
_IN_BOUNDS = lax.GatherScatterMode.PROMISE_IN_BOUNDS
_DNUMS = lax.GatherDimensionNumbers(
    offset_dims=(), collapsed_slice_dims=(0,), start_index_map=(0,))


def _bcast_lane(vec, lidx):
    return lax.gather(vec, lidx[:, None], _DNUMS, (1,), mode=_IN_BOUNDS)


@functools.partial(
    pl.kernel,
    out_type=(
        jax.ShapeDtypeStruct((S, B, D), jnp.float32),
        jax.ShapeDtypeStruct((S, B, D), jnp.float32),
    ),
    mesh=_mesh,
    compiler_params=pltpu.CompilerParams(needs_layout_passes=False),
    scratch_types=[
        pltpu.VMEM((V * D,), jnp.float32),
        pltpu.VMEM((W * B,), jnp.int32),
        pltpu.VMEM((W * B,), jnp.float32),
        pltpu.VMEM((B, D), jnp.float32),
    ],
)
def _encode(prem_idx, hypo_idx, prem_mask, hypo_mask, table_f,
            out_p, out_h, tbl_v, idx_v, mask_v, out_v):
    wid = lax.axis_index("s") * 2 + lax.axis_index("c")
    pltpu.sync_copy(table_f, tbl_v)
    iota = lax.iota(jnp.int32, L)
    iota_c = [iota + c * L for c in range(NC)]

    for idx_hbm, mask_hbm, out_hbm in (
        (prem_idx, prem_mask, out_p),
        (hypo_idx, hypo_mask, out_h),
    ):
        for j in range(SEQ_PER_TILE):
            s = wid * SEQ_PER_TILE + j
            pltpu.sync_copy(idx_hbm.at[s], idx_v)
            pltpu.sync_copy(mask_hbm.at[s], mask_v)

            def bc_body(bc, _):
                iv = [idx_v[pl.ds(w * B + bc * L, L)] for w in range(W)]
                mv = [mask_v[pl.ds(w * B + bc * L, L)] for w in range(W)]

                def l_body(l, _):
                    lidx = jnp.full((L,), l, jnp.int32)
                    b_abs = bc * L + l
                    for w in range(W):
                        rb = _bcast_lane(iv[w], lidx)
                        mb = _bcast_lane(mv[w], lidx)
                        for c in range(NC):
                            g = plsc.load_gather(tbl_v, [rb + iota_c[c]])
                            prod = g * mb
                            dst = out_v.at[b_abs, pl.ds(c * L, L)]
                            if w == 0:
                                out_v[b_abs, pl.ds(c * L, L)] = prod
                            else:
                                plsc.addupdate(dst, prod)
                    return 0

                lax.fori_loop(0, L, l_body, 0)
                return 0

            lax.fori_loop(0, BC, bc_body, 0)
            pltpu.sync_copy(out_v, out_hbm.at[s])


def kernel(char_prem_batch, char_hypo_batch, char_prem_masks, char_hypo_masks,
           table):
    cp = (char_prem_batch.astype(jnp.int32) * D).reshape(S, W * B)
    ch = (char_hypo_batch.astype(jnp.int32) * D).reshape(S, W * B)
    mp = (char_prem_masks * (1.0 / W)).reshape(S, W * B)
    mh = (char_hypo_masks * (1.0 / W)).reshape(S, W * B)
    return _encode(cp, ch, mp, mh, table.reshape(-1))


# speedup vs baseline: 2.6101x; 2.6101x over previous
import functools

import jax
import jax.numpy as jnp
from jax import lax
from jax.experimental import pallas as pl
from jax.experimental.pallas import tpu as pltpu
from jax.experimental.pallas import tpu_sc as plsc

S, W, B, V, D = 128, 16, 256, 1000, 64
L = 16
BC = B // L
SEQ_PER_TILE = 4
NC = D // L

_mesh = plsc.VectorSubcoreMesh(core_axis_name="c", subcore_axis_name="s")
_IN_BOUNDS = lax.GatherScatterMode.PROMISE_IN_BOUNDS
_DNUMS = lax.GatherDimensionNumbers(
    offset_dims=(), collapsed_slice_dims=(0,), start_index_map=(0,))


def _bcast_lane(vec, lidx):
    return lax.gather(vec, lidx[:, None], _DNUMS, (1,), mode=_IN_BOUNDS)


@functools.partial(
    pl.kernel,
    out_type=(
        jax.ShapeDtypeStruct((S, B, D), jnp.float32),
        jax.ShapeDtypeStruct((S, B, D), jnp.float32),
    ),
    mesh=_mesh,
    compiler_params=pltpu.CompilerParams(needs_layout_passes=False),
    scratch_types=[
        pltpu.VMEM((V * D,), jnp.float32),
        pltpu.VMEM((W * B,), jnp.int32),
        pltpu.VMEM((W * B,), jnp.float32),
        pltpu.VMEM((B, D), jnp.float32),
    ],
)
def _encode(prem_idx, hypo_idx, prem_mask, hypo_mask, table_f,
            out_p, out_h, tbl_v, idx_v, mask_v, out_v):
    wid = lax.axis_index("s") * 2 + lax.axis_index("c")
    pltpu.sync_copy(table_f, tbl_v)
    iota = lax.iota(jnp.int32, L)
    iota_c = [iota + c * L for c in range(NC)]

    for idx_hbm, mask_hbm, out_hbm in (
        (prem_idx, prem_mask, out_p),
        (hypo_idx, hypo_mask, out_h),
    ):
        for j in range(SEQ_PER_TILE):
            s = wid * SEQ_PER_TILE + j
            pltpu.sync_copy(idx_hbm.at[s], idx_v)
            pltpu.sync_copy(mask_hbm.at[s], mask_v)

            def bc_body(bc, _):
                iv = [idx_v[pl.ds(w * B + bc * L, L)] for w in range(W)]
                mv = [mask_v[pl.ds(w * B + bc * L, L)] for w in range(W)]

                def l_body(l, _):
                    lidx = jnp.full((L,), l, jnp.int32)
                    b_abs = bc * L + l
                    rb = [_bcast_lane(iv[w], lidx) for w in range(W)]
                    mb = [_bcast_lane(mv[w], lidx) for w in range(W)]
                    for c in range(NC):
                        terms = [
                            plsc.load_gather(tbl_v, [rb[w] + iota_c[c]]) * mb[w]
                            for w in range(W)]
                        while len(terms) > 1:
                            terms = [terms[i] + terms[i + 1]
                                     for i in range(0, len(terms), 2)]
                        out_v[b_abs, pl.ds(c * L, L)] = terms[0]
                    return 0

                lax.fori_loop(0, L, l_body, 0)
                return 0

            lax.fori_loop(0, BC, bc_body, 0)
            pltpu.sync_copy(out_v, out_hbm.at[s])


def kernel(char_prem_batch, char_hypo_batch, char_prem_masks, char_hypo_masks,
           table):
    cp = (char_prem_batch.astype(jnp.int32) * D).reshape(S, W * B)
    ch = (char_hypo_batch.astype(jnp.int32) * D).reshape(S, W * B)
    mp = (char_prem_masks * (1.0 / W)).reshape(S, W * B)
    mh = (char_hypo_masks * (1.0 / W)).reshape(S, W * B)
    return _encode(cp, ch, mp, mh, table.reshape(-1))
